# baseline (speedup 1.0000x reference)
import functools

import jax
import jax.numpy as jnp
from jax import lax
from jax.experimental import pallas as pl
from jax.experimental.pallas import tpu as pltpu
from jax.experimental.pallas import tpu_sc as plsc

BATCH = 16384
EMB = 32
NC = 2
NS = 16
NW = NC * NS
B_PER_W = BATCH // NW
CHUNK = 256

N_ROWS = 1000000
TBLK = 32768
NBLKS = 31
M_ROWS = NBLKS * TBLK // 2


def _tc_mix(ugT, igT, umT, imT):
    def body(a_r, b_r, c_r, d_r, o_r):
        x4 = jnp.concatenate([a_r[...], b_r[...], c_r[...], d_r[...]], axis=0)
        b = jax.lax.bitcast_convert_type(x4, jnp.int32)
        bb = (b + 32767 + ((b >> 16) & 1)) >> 16
        bbT = bb.T
        lo = bbT[0:TBLK // 2]
        hi = bbT[TBLK // 2:TBLK]
        o_r[...] = (lo & 65535) | (hi << 16)

    in_spec = pl.BlockSpec((EMB, TBLK), lambda b: (0, b))
    return pl.pallas_call(
        body,
        grid=(NBLKS,),
        in_specs=[in_spec] * 4,
        out_specs=pl.BlockSpec((TBLK // 2, 128), lambda b: (b, 0)),
        out_shape=jax.ShapeDtypeStruct((M_ROWS, 128), jnp.int32),
        compiler_params=pltpu.CompilerParams(
            dimension_semantics=("arbitrary",),
        ),
    )(ugT, igT, umT, imT)


def _sc_gather(u, i, M):
    mesh = plsc.VectorSubcoreMesh(core_axis_name="c", subcore_axis_name="s")
    out_t = tuple(jax.ShapeDtypeStruct((BATCH, 128), jnp.int32) for _ in range(2))

    @functools.partial(
        pl.kernel,
        mesh=mesh,
        out_type=out_t,
        scratch_types=[
            pltpu.VMEM((B_PER_W,), jnp.int32),
            pltpu.VMEM((B_PER_W,), jnp.int32),
            pltpu.VMEM((CHUNK, 128), jnp.int32),
            pltpu.VMEM((CHUNK, 128), jnp.int32),
            pltpu.SemaphoreType.DMA,
        ],
    )
    def k(u_hbm, i_hbm, m_hbm, o_u, o_i, uidx, iidx, bu, bi, sem):
        wid = lax.axis_index("s") * NC + lax.axis_index("c")
        base = wid * B_PER_W
        pltpu.sync_copy(u_hbm.at[pl.ds(base, B_PER_W)], uidx)
        pltpu.sync_copy(i_hbm.at[pl.ds(base, B_PER_W)], iidx)

        def chunk_body(c, carry):
            coff = c * CHUNK
            c0 = pltpu.async_copy(m_hbm.at[uidx.at[pl.ds(coff, CHUNK)]], bu, sem)
            c1 = pltpu.async_copy(m_hbm.at[iidx.at[pl.ds(coff, CHUNK)]], bi, sem)
            c0.wait()
            c1.wait()
            sl = pl.ds(base + coff, CHUNK)
            pltpu.sync_copy(bu, o_u.at[sl])
            pltpu.sync_copy(bi, o_i.at[sl])
            return carry

        lax.fori_loop(0, B_PER_W // CHUNK, chunk_body, 0)

    return k(u, i, M)


def _tc_dense(Xu, Xi, mu_odd, mi_odd, W1a, W1b, b1, W2, b2, Wfg, Wfh, bf):
    BLK = 2048
    grid = (BATCH // BLK,)

    def unpack(x_i32, odd):
        bits = jnp.where(odd != 0,
                         x_i32 & jnp.int32(-65536),
                         x_i32 << 16)
        return jax.lax.bitcast_convert_type(bits, jnp.float32)

    def body(xu_r, xi_r, mou_r, moi_r, W1a_r, W1b_r, b1_r, W2_r, b2_r,
             Wfg_r, Wfh_r, bf_r, out_r):
        xu = unpack(xu_r[...], mou_r[...])
        xi = unpack(xi_r[...], moi_r[...])
        gu = xu[:, 0:32]
        mu = xu[:, 64:96]
        gi = xi[:, 32:64]
        mi = xi[:, 96:128]
        h = jnp.dot(mu, W1a_r[...], preferred_element_type=jnp.float32)
        h = h + jnp.dot(mi, W1b_r[...], preferred_element_type=jnp.float32)
        h = jnp.maximum(h + b1_r[...], 0.0)
        h2 = jnp.dot(h, W2_r[...], preferred_element_type=jnp.float32)
        h2 = jnp.maximum(h2 + b2_r[...], 0.0)
        gmf = gu * gi
        acc = jnp.sum(gmf * Wfg_r[...], axis=1) + jnp.sum(h2 * Wfh_r[...], axis=1)
        out_r[...] = acc + bf_r[0, 0]

    full = lambda s: pl.BlockSpec(s, lambda b: (0, 0))
    return pl.pallas_call(
        body,
        grid=grid,
        in_specs=[
            pl.BlockSpec((BLK, 128), lambda b: (b, 0)),
            pl.BlockSpec((BLK, 128), lambda b: (b, 0)),
            pl.BlockSpec((BLK, 128), lambda b: (b, 0)),
            pl.BlockSpec((BLK, 128), lambda b: (b, 0)),
            full((EMB, 64)),
            full((EMB, 64)),
            full((1, 64)),
            full((64, EMB)),
            full((1, EMB)),
            full((1, EMB)),
            full((1, EMB)),
            full((1, 1)),
        ],
        out_specs=pl.BlockSpec((BLK,), lambda b: (b,)),
        out_shape=jax.ShapeDtypeStruct((BATCH,), jnp.float32),
        compiler_params=pltpu.CompilerParams(
            dimension_semantics=("parallel",),
        ),
    )(Xu, Xi, mu_odd, mi_odd, W1a, W1b, b1, W2, b2, Wfg, Wfh, bf)


@jax.jit
def kernel(u, i, user_emb_gmf, item_emb_gmf, user_emb_mlp, item_emb_mlp,
           W1, b1, W2, b2, Wf, bf):
    u32 = jnp.asarray(u, jnp.int32)
    i32 = jnp.asarray(i, jnp.int32)

    M = _tc_mix(user_emb_gmf.T, item_emb_gmf.T,
                user_emb_mlp.T, item_emb_mlp.T)
    gu_idx = ((u32 >> 15) << 14) | (u32 & 16383)
    gi_idx = ((i32 >> 15) << 14) | (i32 & 16383)
    Xu, Xi = _sc_gather(gu_idx, gi_idx, M)

    ones = jnp.ones((1, 128), dtype=jnp.int32)
    mu_odd = ((u32 >> 14) & 1)[:, None] * ones
    mi_odd = ((i32 >> 14) & 1)[:, None] * ones

    W1a = W1[:EMB, :]
    W1b = W1[EMB:, :]
    Wfg = Wf[:EMB, 0].reshape(1, EMB)
    Wfh = Wf[EMB:, 0].reshape(1, EMB)
    out = _tc_dense(Xu, Xi, mu_odd, mi_odd, W1a, W1b, b1.reshape(1, 64), W2,
                    b2.reshape(1, EMB), Wfg, Wfh, bf.reshape(1, 1))
    return out
